# Initial kernel scaffold; baseline (speedup 1.0000x reference)
import jax
import jax.numpy as jnp
from jax.experimental import pallas as pl


def kernel(x, edge_index, dist, direction, gat_params, energy_params, force_params, stress_params, u2e_W):
    raise NotImplementedError("write your pallas kernel here")


# speedup vs baseline: 1.0579x; 1.0579x over previous
import jax
import jax.numpy as jnp
from jax.experimental import pallas as pl

N = 10000
E = 320000
D = 128
NEG = 0.2


def _leaky(x):
    return jnp.where(x >= 0, x, NEG * x)


def _readout(t, params, n_act):
    for i, (W, b) in enumerate(params):
        t = t @ W + b
        if i < n_act:
            t = _leaky(t)
    return t


def _id_body(x_ref, o_ref):
    o_ref[...] = x_ref[...]


def kernel(x, edge_index, dist, direction, gat_params, energy_params, force_params, stress_params, u2e_W):
    src = edge_index[0]
    dst = edge_index[1]
    d = jnp.maximum(jnp.reshape(dist, (-1, 1, 1)), 0.01)
    score = 1.0 / d
    h = pl.pallas_call(
        _id_body,
        out_shape=jax.ShapeDtypeStruct((N, D), jnp.float32),
    )(x)
    for (W, b, att_src, att_dst) in gat_params:
        feat = (h @ W + b).reshape(-1, 1, D)
        e_s = jnp.sum(feat * att_src, axis=-1, keepdims=True)
        e_d = jnp.sum(feat * att_dst, axis=-1, keepdims=True)
        e = _leaky(e_s[src] + e_d[dst])
        a = e * score
        m = feat[src] * a
        agg = jax.ops.segment_sum(m, dst, num_segments=N)
        h = _leaky(agg).reshape(-1, D)
    energy_atom = _readout(h, energy_params, 2)
    energy = jnp.mean(energy_atom).reshape(1)
    force_score = ((h[src] + h[dst]).reshape(-1, 1, D) / d).reshape(-1, D)
    force_score = _readout(force_score, force_params, 2)
    fvec = force_score * direction
    force_pred = jax.ops.segment_sum(fvec, dst, num_segments=N)
    u2e = x @ u2e_W
    stress_score = (u2e[src] + u2e[dst]) / d[:, 0, :]
    stress_score = _readout(stress_score, stress_params, 2)
    svec = stress_score * jnp.concatenate([direction, direction], axis=1)
    stress = jnp.mean(svec, axis=0).reshape(1, 6)
    return energy, force_pred, stress


# speedup vs baseline: 4.7613x; 4.5009x over previous
import functools

import jax
import jax.numpy as jnp
from jax import lax
from jax.experimental import pallas as pl
from jax.experimental.pallas import tpu as pltpu
from jax.experimental.pallas import tpu_sc as plsc

N = 10000
E = 320000
D = 128
NEG = 0.2

NC = 2
NS = 16
NW = NC * NS
L = 16
C = 80
EPW = E // NW
NCHUNK = EPW // C
CPW = NCHUNK
FR = 313
NH = 2512
NSWEEP = 2


def _leaky(v):
    return jnp.where(v >= 0, v, NEG * v)


def _dense1_body(x_ref, w_ref, b_ref, asrc_ref, adst_ref, feat_ref, es_ref, ed_ref):
    feat = jnp.dot(x_ref[...], w_ref[...], preferred_element_type=jnp.float32) + b_ref[...]
    feat_ref[...] = feat
    es_ref[...] = jnp.sum(feat * asrc_ref[...], axis=1, keepdims=True)
    ed_ref[...] = jnp.sum(feat * adst_ref[...], axis=1, keepdims=True)


def _dense1(x, w, b, asrc, adst):
    return pl.pallas_call(
        _dense1_body,
        out_shape=[
            jax.ShapeDtypeStruct((N, D), jnp.float32),
            jax.ShapeDtypeStruct((N, 1), jnp.float32),
            jax.ShapeDtypeStruct((N, 1), jnp.float32),
        ],
    )(x, w, b.reshape(1, D), asrc.reshape(1, D), adst.reshape(1, D))


def _dense2_body(agg_ref, w_ref, b_ref, asrc_ref, adst_ref, feat_ref, es_ref, ed_ref):
    h = _leaky(jnp.concatenate([agg_ref[0, 0], agg_ref[0, 1], agg_ref[1, 0],
                                 agg_ref[1, 1, :N - 3 * NH]], axis=0))
    feat = jnp.dot(h, w_ref[...], preferred_element_type=jnp.float32) + b_ref[...]
    feat_ref[...] = feat
    es_ref[...] = jnp.sum(feat * asrc_ref[...], axis=1, keepdims=True)
    ed_ref[...] = jnp.sum(feat * adst_ref[...], axis=1, keepdims=True)


def _dense2(agg, w, b, asrc, adst):
    return pl.pallas_call(
        _dense2_body,
        out_shape=[
            jax.ShapeDtypeStruct((N, D), jnp.float32),
            jax.ShapeDtypeStruct((N, 1), jnp.float32),
            jax.ShapeDtypeStruct((N, 1), jnp.float32),
        ],
    )(agg, w, b.reshape(1, D), asrc.reshape(1, D), adst.reshape(1, D))


def _dense3_body(agg_ref, x_ref, ew0, eb0, ew1, eb1, ew2, eb2, ew3, eb3, ew4, eb4,
                 wf1, u2e, ws1, t_ref, en_ref):
    h = _leaky(jnp.concatenate([agg_ref[0, 0], agg_ref[0, 1], agg_ref[1, 0],
                                 agg_ref[1, 1, :N - 3 * NH]], axis=0))
    t = _leaky(jnp.dot(h, ew0[...], preferred_element_type=jnp.float32) + eb0[...])
    t = _leaky(jnp.dot(t, ew1[...], preferred_element_type=jnp.float32) + eb1[...])
    t = jnp.dot(t, ew2[...], preferred_element_type=jnp.float32) + eb2[...]
    t = jnp.dot(t, ew3[...], preferred_element_type=jnp.float32) + eb3[...]
    t = jnp.dot(t, ew4[...], preferred_element_type=jnp.float32) + eb4[...]
    en_ref[...] = jnp.sum(t, axis=0, keepdims=True) * (1.0 / N)
    gf = jnp.dot(h, wf1[...], preferred_element_type=jnp.float32)
    u2 = jnp.dot(u2e[...], ws1[...], preferred_element_type=jnp.float32)
    gs = jnp.dot(x_ref[...], u2, preferred_element_type=jnp.float32)
    t_ref[...] = jnp.concatenate([gf, gs], axis=1)


def _dense3(agg, x, energy_params, wf1, u2e_w, ws1):
    eflat = []
    for (w, b) in energy_params:
        eflat += [w, b.reshape(1, -1)]
    return pl.pallas_call(
        _dense3_body,
        out_shape=[
            jax.ShapeDtypeStruct((N, D), jnp.float32),
            jax.ShapeDtypeStruct((1, 1), jnp.float32),
        ],
    )(agg, x, *eflat, wf1, u2e_w, ws1)


RBLK = 2000
GSTEPS = E // RBLK


def _readout_body(p_ref, dir_ref,
                  fb1, fw2, fb2, fw3, fb3, fw4, fb4, fw5, fb5,
                  sb1, sw2, sb2, sw3, sb3, sw4, sb4, sw5, sb5,
                  fx_ref, fy_ref, fz_ref, s_ref):
    i = pl.program_id(0)
    p = p_ref[...]
    dirs = dir_ref[...]
    tf = _leaky(p[:, :64] + fb1[...])
    tf = _leaky(jnp.dot(tf, fw2[...], preferred_element_type=jnp.float32) + fb2[...])
    tf = jnp.dot(tf, fw3[...], preferred_element_type=jnp.float32) + fb3[...]
    tf = jnp.dot(tf, fw4[...], preferred_element_type=jnp.float32) + fb4[...]
    tf = jnp.dot(tf, fw5[...], preferred_element_type=jnp.float32) + fb5[...]
    fv = tf * dirs
    fx_ref[...] = fv[:, 0].reshape(1, 1, RBLK)
    fy_ref[...] = fv[:, 1].reshape(1, 1, RBLK)
    fz_ref[...] = fv[:, 2].reshape(1, 1, RBLK)
    ts = _leaky(p[:, 64:] + sb1[...])
    ts = _leaky(jnp.dot(ts, sw2[...], preferred_element_type=jnp.float32) + sb2[...])
    ts = jnp.dot(ts, sw3[...], preferred_element_type=jnp.float32) + sb3[...]
    ts = jnp.dot(ts, sw4[...], preferred_element_type=jnp.float32) + sb4[...]
    ts = jnp.dot(ts, sw5[...], preferred_element_type=jnp.float32) + sb5[...]
    sv = ts * jnp.concatenate([dirs, dirs], axis=1)
    part = jnp.sum(sv, axis=0, keepdims=True)

    @pl.when(i == 0)
    def _():
        s_ref[...] = jnp.zeros_like(s_ref)

    s_ref[...] += part

    @pl.when(i == GSTEPS - 1)
    def _():
        s_ref[...] = s_ref[...] * (1.0 / E)


def _readout(p, direction, force_params, stress_params):
    def flat(params):
        out = [params[0][1].reshape(1, -1)]
        for (w, b) in params[1:]:
            out += [w, b.reshape(1, -1)]
        return out

    wargs = flat(force_params) + flat(stress_params)
    wspecs = [pl.BlockSpec(a.shape, lambda i: (0, 0)) for a in wargs]
    cvec = pl.BlockSpec((1, 1, RBLK), lambda i: (i, 0, 0))
    return pl.pallas_call(
        _readout_body,
        grid=(GSTEPS,),
        in_specs=[
            pl.BlockSpec((RBLK, D), lambda i: (i, 0)),
            pl.BlockSpec((RBLK, 3), lambda i: (i, 0)),
        ] + wspecs,
        out_specs=[cvec, cvec, cvec, pl.BlockSpec((1, 6), lambda i: (0, 0))],
        out_shape=[
            jax.ShapeDtypeStruct((GSTEPS, 1, RBLK), jnp.float32),
            jax.ShapeDtypeStruct((GSTEPS, 1, RBLK), jnp.float32),
            jax.ShapeDtypeStruct((GSTEPS, 1, RBLK), jnp.float32),
            jax.ShapeDtypeStruct((1, 6), jnp.float32),
        ],
    )(p, direction, *wargs)


def _fsum_body(parts_ref, out_ref):
    out_ref[...] = jnp.sum(parts_ref[...], axis=0)


def _fsum(parts):
    return pl.pallas_call(
        _fsum_body,
        out_shape=jax.ShapeDtypeStruct((FR, 128), jnp.float32),
    )(parts)


_SC_MESH = dict(core_axis_name="c", subcore_axis_name="s")

_SPLAT_DNUMS = lax.GatherDimensionNumbers(
    offset_dims=(), collapsed_slice_dims=(0,), start_index_map=(0,))


def _splat(vec, j):
    idx = jnp.full((L, 1), j, jnp.int32)
    return lax.gather(vec, idx, _SPLAT_DNUMS, (1,),
                      mode=lax.GatherScatterMode.PROMISE_IN_BOUNDS)


def _zero_rows(rows, nrow, ncolv):
    def body(r, carry):
        for k in range(ncolv):
            rows[r, pl.ds(k * L, L)] = jnp.zeros((L,), jnp.float32)
        return carry
    lax.fori_loop(0, nrow, body, 0)


def _unit_sweep(s, body):
    upt = -(-NH // 16 // NS)
    nu = jnp.where(s == NS - 1, NH // 16 - upt * (NS - 1), upt)
    ubase = s * upt

    def fbody(u, carry):
        body(ubase + u)
        return carry

    lax.fori_loop(0, nu, fbody, 0)


def _gat_edge(feat, es, ed, src3, dst3, dist3):
    mesh = plsc.VectorSubcoreMesh(**_SC_MESH)

    @functools.partial(
        pl.kernel,
        out_type=jax.ShapeDtypeStruct((NSWEEP, NC, NH, D), jnp.float32),
        mesh=mesh,
        compiler_params=pltpu.CompilerParams(needs_layout_passes=False),
        scratch_types=[
            pltpu.VMEM((N,), jnp.float32),
            pltpu.VMEM((N,), jnp.float32),
            pltpu.VMEM((CPW, C), jnp.int32),
            pltpu.VMEM((CPW, C), jnp.int32),
            pltpu.VMEM((CPW, C), jnp.int32),
            pltpu.VMEM((CPW, C), jnp.float32),
            pltpu.VMEM((C, D), jnp.float32),
            pltpu.VMEM_SHARED((NH, D), jnp.float32),
            pltpu.SemaphoreType.DMA,
        ],
    )
    def k(feat_hbm, es_hbm, ed_hbm, src_hbm, dst_hbm, dist_hbm, out_hbm,
          es_v, ed_v, src_v, dst_v, dloc_v, dist_v, rows, acc, sem):
        c = lax.axis_index("c")
        s = lax.axis_index("s")
        pltpu.sync_copy(es_hbm, es_v)
        pltpu.sync_copy(ed_hbm, ed_v)

        for q in range(NSWEEP):
            nbase = (q * NC + c) * NH
            _zero_rows(rows, 16, D // L)
            _unit_sweep(s, lambda u: pltpu.sync_copy(
                rows.at[pl.ds(0, 16)], acc.at[pl.ds(u * 16, 16)]))
            plsc.subcore_barrier()

            for p in range(2):
                w = s * 2 + p
                pltpu.sync_copy(src_hbm.at[w], src_v)
                pltpu.sync_copy(dst_hbm.at[w], dst_v)
                pltpu.sync_copy(dist_hbm.at[w], dist_v)

                def localize(r, carry):
                    for g in range(C // L):
                        sl = pl.ds(g * L, L)
                        dl = dst_v[r, sl] - nbase
                        ok = (dl >= 0) & (dl < NH)
                        dloc_v[r, sl] = jnp.where(ok, dl, -1)
                    return carry

                lax.fori_loop(0, CPW, localize, 0)

                def chunk(i, carry):
                    pltpu.async_copy(feat_hbm.at[src_v.at[i]], rows, sem).wait()
                    for g in range(C // L):
                        sl = pl.ds(g * L, L)
                        esg = plsc.load_gather(es_v, [src_v[i, sl]])
                        edg = plsc.load_gather(ed_v, [dst_v[i, sl]])
                        dd = jnp.maximum(dist_v[i, sl], 0.01)
                        t = esg + edg
                        a16 = jnp.where(t >= 0, t, NEG * t) / dd
                        for j in range(L):
                            spl = _splat(a16, j)
                            r = g * L + j
                            for k8 in range(D // L):
                                rsl = pl.ds(k8 * L, L)
                                rows[r, rsl] = rows[r, rsl] * spl
                    ix = plsc.Indices(dloc_v.at[i], ignored_value=-1)
                    pltpu.sync_copy(rows, acc.at[ix], add=True)
                    return carry

                lax.fori_loop(0, NCHUNK, chunk, 0)
            plsc.subcore_barrier()

            def copy_out(u):
                pltpu.sync_copy(acc.at[pl.ds(u * 16, 16)], rows.at[pl.ds(0, 16)])
                pltpu.sync_copy(rows.at[pl.ds(0, 16)],
                                out_hbm.at[q, c, pl.ds(u * 16, 16)])

            _unit_sweep(s, copy_out)

    return k(feat, es, ed, src3, dst3, dist3)


def _edge_feat(table, src3, dst3, dist3):
    mesh = plsc.VectorSubcoreMesh(**_SC_MESH)

    @functools.partial(
        pl.kernel,
        out_type=jax.ShapeDtypeStruct((E, D), jnp.float32),
        mesh=mesh,
        compiler_params=pltpu.CompilerParams(needs_layout_passes=False),
        scratch_types=[
            pltpu.VMEM((CPW, C), jnp.int32),
            pltpu.VMEM((CPW, C), jnp.int32),
            pltpu.VMEM((CPW, C), jnp.float32),
            pltpu.VMEM((C, D), jnp.float32),
            pltpu.VMEM((C, D), jnp.float32),
            pltpu.SemaphoreType.DMA,
            pltpu.SemaphoreType.DMA,
        ],
    )
    def k(tab_hbm, src_hbm, dst_hbm, dist_hbm, out_hbm,
          src_v, dst_v, dist_v, rows_s, rows_d, sem1, sem2):
        c = lax.axis_index("c")
        s = lax.axis_index("s")
        w = c * NS + s
        pltpu.sync_copy(src_hbm.at[w], src_v)
        pltpu.sync_copy(dst_hbm.at[w], dst_v)
        pltpu.sync_copy(dist_hbm.at[w], dist_v)
        ebase = w * EPW

        def chunk(i, carry):
            cp1 = pltpu.async_copy(tab_hbm.at[src_v.at[i]], rows_s, sem1)
            cp2 = pltpu.async_copy(tab_hbm.at[dst_v.at[i]], rows_d, sem2)
            cp1.wait()
            cp2.wait()
            for g in range(C // L):
                sl = pl.ds(g * L, L)
                dd = jnp.maximum(dist_v[i, sl], 0.01)
                inv = 1.0 / dd
                for j in range(L):
                    spl = _splat(inv, j)
                    r = g * L + j
                    for k8 in range(D // L):
                        rsl = pl.ds(k8 * L, L)
                        rows_s[r, rsl] = (rows_s[r, rsl] + rows_d[r, rsl]) * spl
            pltpu.sync_copy(rows_s, out_hbm.at[pl.ds(ebase + i * C, C)])
            return carry

        lax.fori_loop(0, NCHUNK, chunk, 0)

    return k(table, src3, dst3, dist3)


def _force_scatter(fx3, fy3, fz3, dst3):
    mesh = plsc.VectorSubcoreMesh(**_SC_MESH)

    @functools.partial(
        pl.kernel,
        out_type=jax.ShapeDtypeStruct((NW, FR, 128), jnp.float32),
        mesh=mesh,
        compiler_params=pltpu.CompilerParams(needs_layout_passes=False),
        scratch_types=[
            pltpu.VMEM((CPW, C), jnp.int32),
            pltpu.VMEM((CPW, C), jnp.float32),
            pltpu.VMEM((CPW, C), jnp.float32),
            pltpu.VMEM((CPW, C), jnp.float32),
            pltpu.VMEM((FR, 128), jnp.float32),
        ],
    )
    def k(fx_hbm, fy_hbm, fz_hbm, dst_hbm, out_hbm, dst_v, fx_v, fy_v, fz_v, acc):
        c = lax.axis_index("c")
        s = lax.axis_index("s")
        w = c * NS + s
        pltpu.sync_copy(dst_hbm.at[w], dst_v)
        pltpu.sync_copy(fx_hbm.at[w], fx_v)
        pltpu.sync_copy(fy_hbm.at[w], fy_v)
        pltpu.sync_copy(fz_hbm.at[w], fz_v)
        _zero_rows(acc, FR, 128 // L)

        def chunk(i, carry):
            for g in range(C // L):
                sl = pl.ds(g * L, L)
                fi = dst_v[i, sl] * 4
                for comp, f_v in ((0, fx_v), (1, fy_v), (2, fz_v)):
                    fic = fi + comp
                    plsc.addupdate_scatter(
                        acc, [lax.shift_right_logical(fic, 7),
                              lax.bitwise_and(fic, 127)],
                        f_v[i, sl])
            return carry

        lax.fori_loop(0, NCHUNK, chunk, 0)
        pltpu.sync_copy(acc, out_hbm.at[w])

    return k(fx3, fy3, fz3, dst3)


def kernel(x, edge_index, dist, direction, gat_params, energy_params,
           force_params, stress_params, u2e_W):
    src3 = edge_index[0].astype(jnp.int32).reshape(NW, CPW, C)
    dst3 = edge_index[1].astype(jnp.int32).reshape(NW, CPW, C)
    dist3 = dist.reshape(NW, CPW, C)

    w1, b1, asrc1, adst1 = gat_params[0]
    w2, b2, asrc2, adst2 = gat_params[1]

    feat1, es1, ed1 = _dense1(x, w1, b1, asrc1, adst1)
    agg1 = _gat_edge(feat1, es1.reshape(N), ed1.reshape(N), src3, dst3, dist3)
    feat2, es2, ed2 = _dense2(agg1, w2, b2, asrc2, adst2)
    agg2 = _gat_edge(feat2, es2.reshape(N), ed2.reshape(N), src3, dst3, dist3)

    table, energy = _dense3(agg2, x, energy_params,
                            force_params[0][0], u2e_W, stress_params[0][0])
    p = _edge_feat(table, src3, dst3, dist3)
    fx, fy, fz, stress = _readout(p, direction, force_params, stress_params)
    parts = _force_scatter(fx.reshape(NW, CPW, C), fy.reshape(NW, CPW, C),
                           fz.reshape(NW, CPW, C), dst3)
    fsum = _fsum(parts)
    force = fsum.reshape(FR * 128)[:N * 4].reshape(N, 4)[:, :3]
    return energy.reshape(1), force, stress


# speedup vs baseline: 7.3484x; 1.5434x over previous
import functools

import jax
import jax.numpy as jnp
from jax import lax
from jax.experimental import pallas as pl
from jax.experimental.pallas import tpu as pltpu
from jax.experimental.pallas import tpu_sc as plsc

N = 10000
E = 320000
D = 128
NEG = 0.2

NC = 2
NS = 16
NW = NC * NS
L = 16
C = 80
EPW = E // NW
NCHUNK = EPW // C
CPW = NCHUNK
FR = 313
NH = 2512
NSWEEP = 2


def _leaky(v):
    return jnp.where(v >= 0, v, NEG * v)


def _dense1_body(x_ref, w_ref, b_ref, asrc_ref, adst_ref, feat_ref, es_ref, ed_ref):
    feat = jnp.dot(x_ref[...], w_ref[...], preferred_element_type=jnp.float32) + b_ref[...]
    feat_ref[...] = feat
    es_ref[...] = jnp.sum(feat * asrc_ref[...], axis=1, keepdims=True)
    ed_ref[...] = jnp.sum(feat * adst_ref[...], axis=1, keepdims=True)


def _dense1(x, w, b, asrc, adst):
    return pl.pallas_call(
        _dense1_body,
        out_shape=[
            jax.ShapeDtypeStruct((N, D), jnp.float32),
            jax.ShapeDtypeStruct((N, 1), jnp.float32),
            jax.ShapeDtypeStruct((N, 1), jnp.float32),
        ],
    )(x, w, b.reshape(1, D), asrc.reshape(1, D), adst.reshape(1, D))


def _dense2_body(agg_ref, w_ref, b_ref, asrc_ref, adst_ref, feat_ref, es_ref, ed_ref):
    h = _leaky(jnp.concatenate([agg_ref[0, 0], agg_ref[0, 1], agg_ref[1, 0],
                                 agg_ref[1, 1, :N - 3 * NH]], axis=0))
    feat = jnp.dot(h, w_ref[...], preferred_element_type=jnp.float32) + b_ref[...]
    feat_ref[...] = feat
    es_ref[...] = jnp.sum(feat * asrc_ref[...], axis=1, keepdims=True)
    ed_ref[...] = jnp.sum(feat * adst_ref[...], axis=1, keepdims=True)


def _dense2(agg, w, b, asrc, adst):
    return pl.pallas_call(
        _dense2_body,
        out_shape=[
            jax.ShapeDtypeStruct((N, D), jnp.float32),
            jax.ShapeDtypeStruct((N, 1), jnp.float32),
            jax.ShapeDtypeStruct((N, 1), jnp.float32),
        ],
    )(agg, w, b.reshape(1, D), asrc.reshape(1, D), adst.reshape(1, D))


def _dense3_body(agg_ref, x_ref, ew0, eb0, ew1, eb1, ew2, eb2, ew3, eb3, ew4, eb4,
                 wf1, u2e, ws1, t_ref, en_ref):
    h = _leaky(jnp.concatenate([agg_ref[0, 0], agg_ref[0, 1], agg_ref[1, 0],
                                 agg_ref[1, 1, :N - 3 * NH]], axis=0))
    t = _leaky(jnp.dot(h, ew0[...], preferred_element_type=jnp.float32) + eb0[...])
    t = _leaky(jnp.dot(t, ew1[...], preferred_element_type=jnp.float32) + eb1[...])
    t = jnp.dot(t, ew2[...], preferred_element_type=jnp.float32) + eb2[...]
    t = jnp.dot(t, ew3[...], preferred_element_type=jnp.float32) + eb3[...]
    t = jnp.dot(t, ew4[...], preferred_element_type=jnp.float32) + eb4[...]
    en_ref[...] = jnp.sum(t, axis=0, keepdims=True) * (1.0 / N)
    gf = jnp.dot(h, wf1[...], preferred_element_type=jnp.float32)
    u2 = jnp.dot(u2e[...], ws1[...], preferred_element_type=jnp.float32)
    gs = jnp.dot(x_ref[...], u2, preferred_element_type=jnp.float32)
    t_ref[...] = jnp.concatenate([gf, gs], axis=1)


def _dense3(agg, x, energy_params, wf1, u2e_w, ws1):
    eflat = []
    for (w, b) in energy_params:
        eflat += [w, b.reshape(1, -1)]
    return pl.pallas_call(
        _dense3_body,
        out_shape=[
            jax.ShapeDtypeStruct((N, D), jnp.float32),
            jax.ShapeDtypeStruct((1, 1), jnp.float32),
        ],
    )(agg, x, *eflat, wf1, u2e_w, ws1)


RBLK = 2000
GSTEPS = E // RBLK


def _readout_body(p_ref, dir_ref,
                  fb1, fw2, fb2, fw3, fb3, fw4, fb4, fw5, fb5,
                  sb1, sw2, sb2, sw3, sb3, sw4, sb4, sw5, sb5,
                  fx_ref, fy_ref, fz_ref, s_ref):
    i = pl.program_id(0)
    p = p_ref[...]
    dirs = dir_ref[...]
    tf = _leaky(p[:, :64] + fb1[...])
    tf = _leaky(jnp.dot(tf, fw2[...], preferred_element_type=jnp.float32) + fb2[...])
    tf = jnp.dot(tf, fw3[...], preferred_element_type=jnp.float32) + fb3[...]
    tf = jnp.dot(tf, fw4[...], preferred_element_type=jnp.float32) + fb4[...]
    tf = jnp.dot(tf, fw5[...], preferred_element_type=jnp.float32) + fb5[...]
    fv = tf * dirs
    fx_ref[...] = fv[:, 0].reshape(1, 1, RBLK)
    fy_ref[...] = fv[:, 1].reshape(1, 1, RBLK)
    fz_ref[...] = fv[:, 2].reshape(1, 1, RBLK)
    ts = _leaky(p[:, 64:] + sb1[...])
    ts = _leaky(jnp.dot(ts, sw2[...], preferred_element_type=jnp.float32) + sb2[...])
    ts = jnp.dot(ts, sw3[...], preferred_element_type=jnp.float32) + sb3[...]
    ts = jnp.dot(ts, sw4[...], preferred_element_type=jnp.float32) + sb4[...]
    ts = jnp.dot(ts, sw5[...], preferred_element_type=jnp.float32) + sb5[...]
    sv = ts * jnp.concatenate([dirs, dirs], axis=1)
    part = jnp.sum(sv, axis=0, keepdims=True)

    @pl.when(i == 0)
    def _():
        s_ref[...] = jnp.zeros_like(s_ref)

    s_ref[...] += part

    @pl.when(i == GSTEPS - 1)
    def _():
        s_ref[...] = s_ref[...] * (1.0 / E)


def _readout(p, direction, force_params, stress_params):
    def flat(params):
        out = [params[0][1].reshape(1, -1)]
        for (w, b) in params[1:]:
            out += [w, b.reshape(1, -1)]
        return out

    wargs = flat(force_params) + flat(stress_params)
    wspecs = [pl.BlockSpec(a.shape, lambda i: (0, 0)) for a in wargs]
    cvec = pl.BlockSpec((1, 1, RBLK), lambda i: (i, 0, 0))
    return pl.pallas_call(
        _readout_body,
        grid=(GSTEPS,),
        in_specs=[
            pl.BlockSpec((RBLK, D), lambda i: (i, 0)),
            pl.BlockSpec((RBLK, 3), lambda i: (i, 0)),
        ] + wspecs,
        out_specs=[cvec, cvec, cvec, pl.BlockSpec((1, 6), lambda i: (0, 0))],
        out_shape=[
            jax.ShapeDtypeStruct((GSTEPS, 1, RBLK), jnp.float32),
            jax.ShapeDtypeStruct((GSTEPS, 1, RBLK), jnp.float32),
            jax.ShapeDtypeStruct((GSTEPS, 1, RBLK), jnp.float32),
            jax.ShapeDtypeStruct((1, 6), jnp.float32),
        ],
    )(p, direction, *wargs)


def _fsum_body(parts_ref, out_ref):
    out_ref[...] = jnp.sum(parts_ref[...], axis=0)


def _fsum(parts):
    return pl.pallas_call(
        _fsum_body,
        out_shape=jax.ShapeDtypeStruct((FR, 128), jnp.float32),
    )(parts)


_SC_MESH = dict(core_axis_name="c", subcore_axis_name="s")

_SPLAT_DNUMS = lax.GatherDimensionNumbers(
    offset_dims=(), collapsed_slice_dims=(0,), start_index_map=(0,))


def _splat(vec, j):
    idx = jnp.full((L, 1), j, jnp.int32)
    return lax.gather(vec, idx, _SPLAT_DNUMS, (1,),
                      mode=lax.GatherScatterMode.PROMISE_IN_BOUNDS)


def _zero_rows(rows, nrow, ncolv):
    def body(r, carry):
        for k in range(ncolv):
            rows[r, pl.ds(k * L, L)] = jnp.zeros((L,), jnp.float32)
        return carry
    lax.fori_loop(0, nrow, body, 0)


def _unit_sweep(s, body):
    upt = -(-NH // 16 // NS)
    nu = jnp.where(s == NS - 1, NH // 16 - upt * (NS - 1), upt)
    ubase = s * upt

    def fbody(u, carry):
        body(ubase + u)
        return carry

    lax.fori_loop(0, nu, fbody, 0)


def _gat_edge(feat, es, ed, src3, dst3, dist3):
    mesh = plsc.VectorSubcoreMesh(**_SC_MESH)

    @functools.partial(
        pl.kernel,
        out_type=jax.ShapeDtypeStruct((NSWEEP, NC, NH, D), jnp.float32),
        mesh=mesh,
        compiler_params=pltpu.CompilerParams(needs_layout_passes=False),
        scratch_types=[
            pltpu.VMEM((N,), jnp.float32),
            pltpu.VMEM((N,), jnp.float32),
            pltpu.VMEM((CPW, C), jnp.int32),
            pltpu.VMEM((CPW, C), jnp.int32),
            pltpu.VMEM((CPW, C), jnp.int32),
            pltpu.VMEM((CPW, C), jnp.float32),
            pltpu.VMEM((C, D), jnp.float32),
            pltpu.VMEM((C, D), jnp.float32),
            pltpu.VMEM_SHARED((NH, D), jnp.float32),
            pltpu.SemaphoreType.DMA,
            pltpu.SemaphoreType.DMA,
        ],
    )
    def k(feat_hbm, es_hbm, ed_hbm, src_hbm, dst_hbm, dist_hbm, out_hbm,
          es_v, ed_v, src_v, dst_v, dloc_v, dist_v, rows, rows1, acc, sem, sem1):
        c = lax.axis_index("c")
        s = lax.axis_index("s")
        pltpu.sync_copy(es_hbm, es_v)
        pltpu.sync_copy(ed_hbm, ed_v)

        def sweep(q, carry0):
            nbase = (q * NC + c) * NH
            _zero_rows(rows, 16, D // L)
            _unit_sweep(s, lambda u: pltpu.sync_copy(
                rows.at[pl.ds(0, 16)], acc.at[pl.ds(u * 16, 16)]))
            plsc.subcore_barrier()

            def phase(p, carry1):
                w = s * 2 + p
                pltpu.sync_copy(src_hbm.at[w], src_v)
                pltpu.sync_copy(dst_hbm.at[w], dst_v)
                pltpu.sync_copy(dist_hbm.at[w], dist_v)

                def localize(r, carry):
                    for g in range(C // L):
                        sl = pl.ds(g * L, L)
                        dl = dst_v[r, sl] - nbase
                        ok = (dl >= 0) & (dl < NH)
                        dloc_v[r, sl] = jnp.where(ok, dl, -1)
                    return carry

                lax.fori_loop(0, CPW, localize, 0)

                def work(i, buf):
                    def group(g, carry):
                        sl = pl.ds(g * L, L)
                        esg = plsc.load_gather(es_v, [src_v[i, sl]])
                        edg = plsc.load_gather(ed_v, [dst_v[i, sl]])
                        dd = jnp.maximum(dist_v[i, sl], 0.01)
                        t = esg + edg
                        a16 = jnp.where(t >= 0, t, NEG * t) / dd
                        for j in range(L):
                            spl = _splat(a16, j)
                            r = g * L + j
                            for k8 in range(D // L):
                                rsl = pl.ds(k8 * L, L)
                                buf[r, rsl] = buf[r, rsl] * spl
                        return carry

                    lax.fori_loop(0, C // L, group, 0)
                    ix = plsc.Indices(dloc_v.at[i], ignored_value=-1)
                    pltpu.sync_copy(buf, acc.at[ix], add=True)

                pltpu.async_copy(feat_hbm.at[src_v.at[0]], rows, sem)

                def pair(t, carry):
                    i0 = 2 * t
                    pltpu.make_async_copy(
                        feat_hbm.at[src_v.at[i0]], rows, sem).wait()
                    pltpu.async_copy(
                        feat_hbm.at[src_v.at[i0 + 1]], rows1, sem1)
                    work(i0, rows)
                    pltpu.make_async_copy(
                        feat_hbm.at[src_v.at[i0 + 1]], rows1, sem1).wait()
                    pltpu.async_copy(
                        feat_hbm.at[src_v.at[i0 + 2]], rows, sem)
                    work(i0 + 1, rows1)
                    return carry

                lax.fori_loop(0, NCHUNK // 2, pair, 0)
                pltpu.make_async_copy(
                    feat_hbm.at[src_v.at[NCHUNK - 1]], rows, sem).wait()
                work(NCHUNK - 1, rows)
                return carry1

            lax.fori_loop(0, 2, phase, 0)
            plsc.subcore_barrier()

            def copy_out(u):
                pltpu.sync_copy(acc.at[pl.ds(u * 16, 16)], rows.at[pl.ds(0, 16)])
                pltpu.sync_copy(rows.at[pl.ds(0, 16)],
                                out_hbm.at[q, c, pl.ds(u * 16, 16)])

            _unit_sweep(s, copy_out)
            return carry0

        lax.fori_loop(0, NSWEEP, sweep, 0)

    return k(feat, es, ed, src3, dst3, dist3)


def _edge_feat(table, src3, dst3, dist3):
    mesh = plsc.VectorSubcoreMesh(**_SC_MESH)

    @functools.partial(
        pl.kernel,
        out_type=jax.ShapeDtypeStruct((E, D), jnp.float32),
        mesh=mesh,
        compiler_params=pltpu.CompilerParams(needs_layout_passes=False),
        scratch_types=[
            pltpu.VMEM((CPW, C), jnp.int32),
            pltpu.VMEM((CPW, C), jnp.int32),
            pltpu.VMEM((CPW, C), jnp.float32),
            pltpu.VMEM((C, D), jnp.float32),
            pltpu.VMEM((C, D), jnp.float32),
            pltpu.VMEM((C, D), jnp.float32),
            pltpu.VMEM((C, D), jnp.float32),
            pltpu.SemaphoreType.DMA,
            pltpu.SemaphoreType.DMA,
            pltpu.SemaphoreType.DMA,
            pltpu.SemaphoreType.DMA,
        ],
    )
    def k(tab_hbm, src_hbm, dst_hbm, dist_hbm, out_hbm,
          src_v, dst_v, dist_v, rows_s, rows_d, rows_s1, rows_d1,
          sem1, sem2, sem3, sem4):
        c = lax.axis_index("c")
        s = lax.axis_index("s")
        w = c * NS + s
        pltpu.sync_copy(src_hbm.at[w], src_v)
        pltpu.sync_copy(dst_hbm.at[w], dst_v)
        pltpu.sync_copy(dist_hbm.at[w], dist_v)
        ebase = w * EPW

        def issue(i, bs, bd, ss, sd):
            pltpu.async_copy(tab_hbm.at[src_v.at[i]], bs, ss)
            pltpu.async_copy(tab_hbm.at[dst_v.at[i]], bd, sd)

        def work(i, bs, bd, ss, sd):
            pltpu.make_async_copy(tab_hbm.at[src_v.at[i]], bs, ss).wait()
            pltpu.make_async_copy(tab_hbm.at[dst_v.at[i]], bd, sd).wait()
            def group(g, carry):
                sl = pl.ds(g * L, L)
                dd = jnp.maximum(dist_v[i, sl], 0.01)
                inv = 1.0 / dd
                for j in range(L):
                    spl = _splat(inv, j)
                    r = g * L + j
                    for k8 in range(D // L):
                        rsl = pl.ds(k8 * L, L)
                        bs[r, rsl] = (bs[r, rsl] + bd[r, rsl]) * spl
                return carry

            lax.fori_loop(0, C // L, group, 0)
            pltpu.sync_copy(bs, out_hbm.at[pl.ds(ebase + i * C, C)])

        issue(0, rows_s, rows_d, sem1, sem2)

        def pair(t, carry):
            i0 = 2 * t
            issue(i0 + 1, rows_s1, rows_d1, sem3, sem4)
            work(i0, rows_s, rows_d, sem1, sem2)
            issue(i0 + 2, rows_s, rows_d, sem1, sem2)
            work(i0 + 1, rows_s1, rows_d1, sem3, sem4)
            return carry

        lax.fori_loop(0, NCHUNK // 2, pair, 0)
        work(NCHUNK - 1, rows_s, rows_d, sem1, sem2)

    return k(table, src3, dst3, dist3)


def _force_scatter(fx3, fy3, fz3, dst3):
    mesh = plsc.VectorSubcoreMesh(**_SC_MESH)

    @functools.partial(
        pl.kernel,
        out_type=jax.ShapeDtypeStruct((NW, FR, 128), jnp.float32),
        mesh=mesh,
        compiler_params=pltpu.CompilerParams(needs_layout_passes=False),
        scratch_types=[
            pltpu.VMEM((CPW, C), jnp.int32),
            pltpu.VMEM((CPW, C), jnp.float32),
            pltpu.VMEM((CPW, C), jnp.float32),
            pltpu.VMEM((CPW, C), jnp.float32),
            pltpu.VMEM((FR, 128), jnp.float32),
        ],
    )
    def k(fx_hbm, fy_hbm, fz_hbm, dst_hbm, out_hbm, dst_v, fx_v, fy_v, fz_v, acc):
        c = lax.axis_index("c")
        s = lax.axis_index("s")
        w = c * NS + s
        pltpu.sync_copy(dst_hbm.at[w], dst_v)
        pltpu.sync_copy(fx_hbm.at[w], fx_v)
        pltpu.sync_copy(fy_hbm.at[w], fy_v)
        pltpu.sync_copy(fz_hbm.at[w], fz_v)
        _zero_rows(acc, FR, 128 // L)

        def chunk(i, carry):
            for g in range(C // L):
                sl = pl.ds(g * L, L)
                fi = dst_v[i, sl] * 4
                for comp, f_v in ((0, fx_v), (1, fy_v), (2, fz_v)):
                    fic = fi + comp
                    plsc.addupdate_scatter(
                        acc, [lax.shift_right_logical(fic, 7),
                              lax.bitwise_and(fic, 127)],
                        f_v[i, sl])
            return carry

        lax.fori_loop(0, NCHUNK, chunk, 0)
        pltpu.sync_copy(acc, out_hbm.at[w])

    return k(fx3, fy3, fz3, dst3)


def kernel(x, edge_index, dist, direction, gat_params, energy_params,
           force_params, stress_params, u2e_W):
    src3 = edge_index[0].astype(jnp.int32).reshape(NW, CPW, C)
    dst3 = edge_index[1].astype(jnp.int32).reshape(NW, CPW, C)
    dist3 = dist.reshape(NW, CPW, C)

    w1, b1, asrc1, adst1 = gat_params[0]
    w2, b2, asrc2, adst2 = gat_params[1]

    feat1, es1, ed1 = _dense1(x, w1, b1, asrc1, adst1)
    agg1 = _gat_edge(feat1, es1.reshape(N), ed1.reshape(N), src3, dst3, dist3)
    feat2, es2, ed2 = _dense2(agg1, w2, b2, asrc2, adst2)
    agg2 = _gat_edge(feat2, es2.reshape(N), ed2.reshape(N), src3, dst3, dist3)

    table, energy = _dense3(agg2, x, energy_params,
                            force_params[0][0], u2e_W, stress_params[0][0])
    p = _edge_feat(table, src3, dst3, dist3)
    fx, fy, fz, stress = _readout(p, direction, force_params, stress_params)
    parts = _force_scatter(fx.reshape(NW, CPW, C), fy.reshape(NW, CPW, C),
                           fz.reshape(NW, CPW, C), dst3)
    fsum = _fsum(parts)
    force = fsum.reshape(FR * 128)[:N * 4].reshape(N, 4)[:, :3]
    return energy.reshape(1), force, stress


# speedup vs baseline: 7.4759x; 1.0173x over previous
import functools

import jax
import jax.numpy as jnp
from jax import lax
from jax.experimental import pallas as pl
from jax.experimental.pallas import tpu as pltpu
from jax.experimental.pallas import tpu_sc as plsc

N = 10000
E = 320000
D = 128
NEG = 0.2

NC = 2
NS = 16
NW = NC * NS
L = 16
C = 80
EPW = E // NW
NCHUNK = EPW // C
CPW = NCHUNK
FR = 313
NH = 2512
NSWEEP = 2


def _leaky(v):
    return jnp.where(v >= 0, v, NEG * v)


def _dense1_body(x_ref, w_ref, b_ref, asrc_ref, adst_ref, feat_ref, es_ref, ed_ref):
    feat = jnp.dot(x_ref[...], w_ref[...], preferred_element_type=jnp.float32) + b_ref[...]
    feat_ref[...] = feat
    es_ref[...] = jnp.sum(feat * asrc_ref[...], axis=1, keepdims=True)
    ed_ref[...] = jnp.sum(feat * adst_ref[...], axis=1, keepdims=True)


def _dense1(x, w, b, asrc, adst):
    return pl.pallas_call(
        _dense1_body,
        out_shape=[
            jax.ShapeDtypeStruct((N, D), jnp.float32),
            jax.ShapeDtypeStruct((N, 1), jnp.float32),
            jax.ShapeDtypeStruct((N, 1), jnp.float32),
        ],
    )(x, w, b.reshape(1, D), asrc.reshape(1, D), adst.reshape(1, D))


def _dense2_body(agg_ref, w_ref, b_ref, asrc_ref, adst_ref, feat_ref, es_ref, ed_ref):
    h = _leaky(jnp.concatenate([agg_ref[0, 0], agg_ref[0, 1], agg_ref[1, 0],
                                 agg_ref[1, 1, :N - 3 * NH]], axis=0))
    feat = jnp.dot(h, w_ref[...], preferred_element_type=jnp.float32) + b_ref[...]
    feat_ref[...] = feat
    es_ref[...] = jnp.sum(feat * asrc_ref[...], axis=1, keepdims=True)
    ed_ref[...] = jnp.sum(feat * adst_ref[...], axis=1, keepdims=True)


def _dense2(agg, w, b, asrc, adst):
    return pl.pallas_call(
        _dense2_body,
        out_shape=[
            jax.ShapeDtypeStruct((N, D), jnp.float32),
            jax.ShapeDtypeStruct((N, 1), jnp.float32),
            jax.ShapeDtypeStruct((N, 1), jnp.float32),
        ],
    )(agg, w, b.reshape(1, D), asrc.reshape(1, D), adst.reshape(1, D))


def _dense3_body(agg_ref, x_ref, ew0, eb0, ew1, eb1, ew2, eb2, ew3, eb3, ew4, eb4,
                 wf1, u2e, ws1, t_ref, en_ref):
    h = _leaky(jnp.concatenate([agg_ref[0, 0], agg_ref[0, 1], agg_ref[1, 0],
                                 agg_ref[1, 1, :N - 3 * NH]], axis=0))
    t = _leaky(jnp.dot(h, ew0[...], preferred_element_type=jnp.float32) + eb0[...])
    t = _leaky(jnp.dot(t, ew1[...], preferred_element_type=jnp.float32) + eb1[...])
    t = jnp.dot(t, ew2[...], preferred_element_type=jnp.float32) + eb2[...]
    t = jnp.dot(t, ew3[...], preferred_element_type=jnp.float32) + eb3[...]
    t = jnp.dot(t, ew4[...], preferred_element_type=jnp.float32) + eb4[...]
    en_ref[...] = jnp.sum(t, axis=0, keepdims=True) * (1.0 / N)
    gf = jnp.dot(h, wf1[...], preferred_element_type=jnp.float32)
    u2 = jnp.dot(u2e[...], ws1[...], preferred_element_type=jnp.float32)
    gs = jnp.dot(x_ref[...], u2, preferred_element_type=jnp.float32)
    t_ref[...] = jnp.concatenate([gf, gs], axis=1)


def _dense3(agg, x, energy_params, wf1, u2e_w, ws1):
    eflat = []
    for (w, b) in energy_params:
        eflat += [w, b.reshape(1, -1)]
    return pl.pallas_call(
        _dense3_body,
        out_shape=[
            jax.ShapeDtypeStruct((N, D), jnp.float32),
            jax.ShapeDtypeStruct((1, 1), jnp.float32),
        ],
    )(agg, x, *eflat, wf1, u2e_w, ws1)


RBLK = 8000
GSTEPS = E // RBLK


def _readout_body(p_ref, dir_ref,
                  fb1, fw2, fb2, fw3, fb3, fw4, fb4, fw5, fb5,
                  sb1, sw2, sb2, sw3, sb3, sw4, sb4, sw5, sb5,
                  fx_ref, fy_ref, fz_ref, s_ref):
    i = pl.program_id(0)
    p = p_ref[...]
    dirs = dir_ref[...]
    tf = _leaky(p[:, :64] + fb1[...])
    tf = _leaky(jnp.dot(tf, fw2[...], preferred_element_type=jnp.float32) + fb2[...])
    tf = jnp.dot(tf, fw3[...], preferred_element_type=jnp.float32) + fb3[...]
    tf = jnp.dot(tf, fw4[...], preferred_element_type=jnp.float32) + fb4[...]
    tf = jnp.dot(tf, fw5[...], preferred_element_type=jnp.float32) + fb5[...]
    fv = tf * dirs
    fx_ref[...] = fv[:, 0].reshape(1, 1, RBLK)
    fy_ref[...] = fv[:, 1].reshape(1, 1, RBLK)
    fz_ref[...] = fv[:, 2].reshape(1, 1, RBLK)
    ts = _leaky(p[:, 64:] + sb1[...])
    ts = _leaky(jnp.dot(ts, sw2[...], preferred_element_type=jnp.float32) + sb2[...])
    ts = jnp.dot(ts, sw3[...], preferred_element_type=jnp.float32) + sb3[...]
    ts = jnp.dot(ts, sw4[...], preferred_element_type=jnp.float32) + sb4[...]
    ts = jnp.dot(ts, sw5[...], preferred_element_type=jnp.float32) + sb5[...]
    sv = ts * jnp.concatenate([dirs, dirs], axis=1)
    part = jnp.sum(sv, axis=0, keepdims=True)

    @pl.when(i == 0)
    def _():
        s_ref[...] = jnp.zeros_like(s_ref)

    s_ref[...] += part

    @pl.when(i == GSTEPS - 1)
    def _():
        s_ref[...] = s_ref[...] * (1.0 / E)


def _readout(p, direction, force_params, stress_params):
    def flat(params):
        out = [params[0][1].reshape(1, -1)]
        for (w, b) in params[1:]:
            out += [w, b.reshape(1, -1)]
        return out

    wargs = flat(force_params) + flat(stress_params)
    wspecs = [pl.BlockSpec(a.shape, lambda i: (0, 0)) for a in wargs]
    cvec = pl.BlockSpec((1, 1, RBLK), lambda i: (i, 0, 0))
    return pl.pallas_call(
        _readout_body,
        grid=(GSTEPS,),
        in_specs=[
            pl.BlockSpec((RBLK, D), lambda i: (i, 0)),
            pl.BlockSpec((RBLK, 3), lambda i: (i, 0)),
        ] + wspecs,
        out_specs=[cvec, cvec, cvec, pl.BlockSpec((1, 6), lambda i: (0, 0))],
        out_shape=[
            jax.ShapeDtypeStruct((GSTEPS, 1, RBLK), jnp.float32),
            jax.ShapeDtypeStruct((GSTEPS, 1, RBLK), jnp.float32),
            jax.ShapeDtypeStruct((GSTEPS, 1, RBLK), jnp.float32),
            jax.ShapeDtypeStruct((1, 6), jnp.float32),
        ],
    )(p, direction, *wargs)


def _fsum_body(parts_ref, out_ref):
    out_ref[...] = jnp.sum(parts_ref[...], axis=0)


def _fsum(parts):
    return pl.pallas_call(
        _fsum_body,
        out_shape=jax.ShapeDtypeStruct((FR, 128), jnp.float32),
    )(parts)


_SC_MESH = dict(core_axis_name="c", subcore_axis_name="s")

_SPLAT_DNUMS = lax.GatherDimensionNumbers(
    offset_dims=(), collapsed_slice_dims=(0,), start_index_map=(0,))


def _splat(vec, j):
    idx = jnp.full((L, 1), j, jnp.int32)
    return lax.gather(vec, idx, _SPLAT_DNUMS, (1,),
                      mode=lax.GatherScatterMode.PROMISE_IN_BOUNDS)


def _zero_rows(rows, nrow, ncolv):
    def body(r, carry):
        for k in range(ncolv):
            rows[r, pl.ds(k * L, L)] = jnp.zeros((L,), jnp.float32)
        return carry
    lax.fori_loop(0, nrow, body, 0)


def _unit_sweep(s, body):
    upt = -(-NH // 16 // NS)
    nu = jnp.where(s == NS - 1, NH // 16 - upt * (NS - 1), upt)
    ubase = s * upt

    def fbody(u, carry):
        body(ubase + u)
        return carry

    lax.fori_loop(0, nu, fbody, 0)


def _gat_edge(feat, es, ed, src3, dst3, dist3):
    mesh = plsc.VectorSubcoreMesh(**_SC_MESH)

    @functools.partial(
        pl.kernel,
        out_type=jax.ShapeDtypeStruct((NSWEEP, NC, NH, D), jnp.float32),
        mesh=mesh,
        compiler_params=pltpu.CompilerParams(needs_layout_passes=False),
        scratch_types=[
            pltpu.VMEM((N,), jnp.float32),
            pltpu.VMEM((N,), jnp.float32),
            pltpu.VMEM((CPW, C), jnp.int32),
            pltpu.VMEM((CPW, C), jnp.int32),
            pltpu.VMEM((CPW, C), jnp.int32),
            pltpu.VMEM((CPW, C), jnp.float32),
            pltpu.VMEM((C, D), jnp.float32),
            pltpu.VMEM((C, D), jnp.float32),
            pltpu.VMEM_SHARED((NH, D), jnp.float32),
            pltpu.SemaphoreType.DMA,
            pltpu.SemaphoreType.DMA,
        ],
    )
    def k(feat_hbm, es_hbm, ed_hbm, src_hbm, dst_hbm, dist_hbm, out_hbm,
          es_v, ed_v, src_v, dst_v, dloc_v, dist_v, rows, rows1, acc, sem, sem1):
        c = lax.axis_index("c")
        s = lax.axis_index("s")
        pltpu.sync_copy(es_hbm, es_v)
        pltpu.sync_copy(ed_hbm, ed_v)

        def sweep(q, carry0):
            nbase = (q * NC + c) * NH
            _zero_rows(rows, 16, D // L)
            _unit_sweep(s, lambda u: pltpu.sync_copy(
                rows.at[pl.ds(0, 16)], acc.at[pl.ds(u * 16, 16)]))
            plsc.subcore_barrier()

            def phase(p, carry1):
                w = s * 2 + p
                pltpu.sync_copy(src_hbm.at[w], src_v)
                pltpu.sync_copy(dst_hbm.at[w], dst_v)
                pltpu.sync_copy(dist_hbm.at[w], dist_v)

                def localize(r, carry):
                    for g in range(C // L):
                        sl = pl.ds(g * L, L)
                        dl = dst_v[r, sl] - nbase
                        ok = (dl >= 0) & (dl < NH)
                        dloc_v[r, sl] = jnp.where(ok, dl, -1)
                    return carry

                lax.fori_loop(0, CPW, localize, 0)

                def work(i, buf):
                    def group(g, carry):
                        sl = pl.ds(g * L, L)
                        esg = plsc.load_gather(es_v, [src_v[i, sl]])
                        edg = plsc.load_gather(ed_v, [dst_v[i, sl]])
                        dd = jnp.maximum(dist_v[i, sl], 0.01)
                        t = esg + edg
                        a16 = jnp.where(t >= 0, t, NEG * t) / dd
                        for j in range(L):
                            spl = _splat(a16, j)
                            r = g * L + j
                            for k8 in range(D // L):
                                rsl = pl.ds(k8 * L, L)
                                buf[r, rsl] = buf[r, rsl] * spl
                        return carry

                    lax.fori_loop(0, C // L, group, 0)
                    ix = plsc.Indices(dloc_v.at[i], ignored_value=-1)
                    pltpu.sync_copy(buf, acc.at[ix], add=True)

                pltpu.async_copy(feat_hbm.at[src_v.at[0]], rows, sem)

                def pair(t, carry):
                    i0 = 2 * t
                    pltpu.make_async_copy(
                        feat_hbm.at[src_v.at[i0]], rows, sem).wait()
                    pltpu.async_copy(
                        feat_hbm.at[src_v.at[i0 + 1]], rows1, sem1)
                    work(i0, rows)
                    pltpu.make_async_copy(
                        feat_hbm.at[src_v.at[i0 + 1]], rows1, sem1).wait()
                    pltpu.async_copy(
                        feat_hbm.at[src_v.at[i0 + 2]], rows, sem)
                    work(i0 + 1, rows1)
                    return carry

                lax.fori_loop(0, NCHUNK // 2, pair, 0)
                pltpu.make_async_copy(
                    feat_hbm.at[src_v.at[NCHUNK - 1]], rows, sem).wait()
                work(NCHUNK - 1, rows)
                return carry1

            lax.fori_loop(0, 2, phase, 0)
            plsc.subcore_barrier()

            def copy_out(u):
                pltpu.sync_copy(acc.at[pl.ds(u * 16, 16)], rows.at[pl.ds(0, 16)])
                pltpu.sync_copy(rows.at[pl.ds(0, 16)],
                                out_hbm.at[q, c, pl.ds(u * 16, 16)])

            _unit_sweep(s, copy_out)
            return carry0

        lax.fori_loop(0, NSWEEP, sweep, 0)

    return k(feat, es, ed, src3, dst3, dist3)


def _edge_feat(table, src3, dst3, dist3):
    mesh = plsc.VectorSubcoreMesh(**_SC_MESH)

    @functools.partial(
        pl.kernel,
        out_type=jax.ShapeDtypeStruct((E, D), jnp.float32),
        mesh=mesh,
        compiler_params=pltpu.CompilerParams(needs_layout_passes=False),
        scratch_types=[
            pltpu.VMEM((CPW, C), jnp.int32),
            pltpu.VMEM((CPW, C), jnp.int32),
            pltpu.VMEM((CPW, C), jnp.float32),
            pltpu.VMEM((C, D), jnp.float32),
            pltpu.VMEM((C, D), jnp.float32),
            pltpu.VMEM((C, D), jnp.float32),
            pltpu.VMEM((C, D), jnp.float32),
            pltpu.SemaphoreType.DMA,
            pltpu.SemaphoreType.DMA,
            pltpu.SemaphoreType.DMA,
            pltpu.SemaphoreType.DMA,
        ],
    )
    def k(tab_hbm, src_hbm, dst_hbm, dist_hbm, out_hbm,
          src_v, dst_v, dist_v, rows_s, rows_d, rows_s1, rows_d1,
          sem1, sem2, sem3, sem4):
        c = lax.axis_index("c")
        s = lax.axis_index("s")
        w = c * NS + s
        pltpu.sync_copy(src_hbm.at[w], src_v)
        pltpu.sync_copy(dst_hbm.at[w], dst_v)
        pltpu.sync_copy(dist_hbm.at[w], dist_v)
        ebase = w * EPW

        def issue(i, bs, bd, ss, sd):
            pltpu.async_copy(tab_hbm.at[src_v.at[i]], bs, ss)
            pltpu.async_copy(tab_hbm.at[dst_v.at[i]], bd, sd)

        def work(i, bs, bd, ss, sd):
            pltpu.make_async_copy(tab_hbm.at[src_v.at[i]], bs, ss).wait()
            pltpu.make_async_copy(tab_hbm.at[dst_v.at[i]], bd, sd).wait()
            def group(g, carry):
                sl = pl.ds(g * L, L)
                dd = jnp.maximum(dist_v[i, sl], 0.01)
                inv = 1.0 / dd
                for j in range(L):
                    spl = _splat(inv, j)
                    r = g * L + j
                    for k8 in range(D // L):
                        rsl = pl.ds(k8 * L, L)
                        bs[r, rsl] = (bs[r, rsl] + bd[r, rsl]) * spl
                return carry

            lax.fori_loop(0, C // L, group, 0)
            pltpu.sync_copy(bs, out_hbm.at[pl.ds(ebase + i * C, C)])

        issue(0, rows_s, rows_d, sem1, sem2)

        def pair(t, carry):
            i0 = 2 * t
            issue(i0 + 1, rows_s1, rows_d1, sem3, sem4)
            work(i0, rows_s, rows_d, sem1, sem2)
            issue(i0 + 2, rows_s, rows_d, sem1, sem2)
            work(i0 + 1, rows_s1, rows_d1, sem3, sem4)
            return carry

        lax.fori_loop(0, NCHUNK // 2, pair, 0)
        work(NCHUNK - 1, rows_s, rows_d, sem1, sem2)

    return k(table, src3, dst3, dist3)


def _force_scatter(fx3, fy3, fz3, dst3):
    mesh = plsc.VectorSubcoreMesh(**_SC_MESH)

    @functools.partial(
        pl.kernel,
        out_type=jax.ShapeDtypeStruct((NW, FR, 128), jnp.float32),
        mesh=mesh,
        compiler_params=pltpu.CompilerParams(needs_layout_passes=False),
        scratch_types=[
            pltpu.VMEM((CPW, C), jnp.int32),
            pltpu.VMEM((CPW, C), jnp.float32),
            pltpu.VMEM((CPW, C), jnp.float32),
            pltpu.VMEM((CPW, C), jnp.float32),
            pltpu.VMEM((FR, 128), jnp.float32),
        ],
    )
    def k(fx_hbm, fy_hbm, fz_hbm, dst_hbm, out_hbm, dst_v, fx_v, fy_v, fz_v, acc):
        c = lax.axis_index("c")
        s = lax.axis_index("s")
        w = c * NS + s
        pltpu.sync_copy(dst_hbm.at[w], dst_v)
        pltpu.sync_copy(fx_hbm.at[w], fx_v)
        pltpu.sync_copy(fy_hbm.at[w], fy_v)
        pltpu.sync_copy(fz_hbm.at[w], fz_v)
        _zero_rows(acc, FR, 128 // L)

        def chunk(i, carry):
            for g in range(C // L):
                sl = pl.ds(g * L, L)
                fi = dst_v[i, sl] * 4
                for comp, f_v in ((0, fx_v), (1, fy_v), (2, fz_v)):
                    fic = fi + comp
                    plsc.addupdate_scatter(
                        acc, [lax.shift_right_logical(fic, 7),
                              lax.bitwise_and(fic, 127)],
                        f_v[i, sl])
            return carry

        lax.fori_loop(0, NCHUNK, chunk, 0)
        pltpu.sync_copy(acc, out_hbm.at[w])

    return k(fx3, fy3, fz3, dst3)


def kernel(x, edge_index, dist, direction, gat_params, energy_params,
           force_params, stress_params, u2e_W):
    src3 = edge_index[0].astype(jnp.int32).reshape(NW, CPW, C)
    dst3 = edge_index[1].astype(jnp.int32).reshape(NW, CPW, C)
    dist3 = dist.reshape(NW, CPW, C)

    w1, b1, asrc1, adst1 = gat_params[0]
    w2, b2, asrc2, adst2 = gat_params[1]

    feat1, es1, ed1 = _dense1(x, w1, b1, asrc1, adst1)
    agg1 = _gat_edge(feat1, es1.reshape(N), ed1.reshape(N), src3, dst3, dist3)
    feat2, es2, ed2 = _dense2(agg1, w2, b2, asrc2, adst2)
    agg2 = _gat_edge(feat2, es2.reshape(N), ed2.reshape(N), src3, dst3, dist3)

    table, energy = _dense3(agg2, x, energy_params,
                            force_params[0][0], u2e_W, stress_params[0][0])
    p = _edge_feat(table, src3, dst3, dist3)
    fx, fy, fz, stress = _readout(p, direction, force_params, stress_params)
    parts = _force_scatter(fx.reshape(NW, CPW, C), fy.reshape(NW, CPW, C),
                           fz.reshape(NW, CPW, C), dst3)
    fsum = _fsum(parts)
    force = fsum.reshape(FR * 128)[:N * 4].reshape(N, 4)[:, :3]
    return energy.reshape(1), force, stress
